# baseline (speedup 1.0000x reference)
import functools

import jax
import jax.numpy as jnp
from jax import lax
from jax.experimental import pallas as pl
from jax.experimental.pallas import tpu as pltpu
from jax.experimental.pallas import tpu_sc as plsc

N = 10000
E = 160000
T = 256
H = 50

NC = 2
NS = 16
L = 16

NP = 10240
EP = 163840
HALF = NP // NC
TRASH = 256
ACC_ROWS = HALF + TRASH
ZROW = NP - 1
CH = 128

_f32 = jnp.float32


def _deg_body(dst_hbm, dega_hbm, degb_hbm, idx_v, ones_v, zslice_v, deg_sh):
    c = lax.axis_index("c")
    s = lax.axis_index("s")
    nslice = NP // NS

    for j in range(CH // L):
        ones_v[pl.ds(j * L, L)] = jnp.ones((L,), _f32)

    def zbody(i, carry):
        zslice_v[pl.ds(i * L, L)] = jnp.zeros((L,), _f32)
        return carry

    lax.fori_loop(0, nslice // L, zbody, 0)
    pltpu.sync_copy(zslice_v, deg_sh.at[pl.ds(s * nslice, nslice)])
    plsc.subcore_barrier()

    ept = EP // (NC * NS)

    def body(i, carry):
        base = (c * NS + s) * ept + i * CH
        pltpu.sync_copy(dst_hbm.at[pl.ds(base, CH)], idx_v)
        pltpu.sync_copy(ones_v, deg_sh.at[idx_v], add=True)
        return carry

    lax.fori_loop(0, ept // CH, body, 0)
    plsc.subcore_barrier()

    @pl.when(c == 0)
    def _():
        pltpu.sync_copy(
            deg_sh.at[pl.ds(s * nslice, nslice)],
            dega_hbm.at[pl.ds(s * nslice, nslice)],
        )

    @pl.when(c == 1)
    def _():
        pltpu.sync_copy(
            deg_sh.at[pl.ds(s * nslice, nslice)],
            degb_hbm.at[pl.ds(s * nslice, nslice)],
        )


def _agg_body(xws_hbm, src_hbm, dst_hbm, out_hbm, src_v, dst_v, rows_v, z16_v, acc_sh):
    c = lax.axis_index("c")
    s = lax.axis_index("s")

    for i in range(16):
        for h in range(2):
            for j in range(128 // L):
                z16_v[i, h, pl.ds(j * L, L)] = jnp.zeros((L,), _f32)

    rpt = ACC_ROWS // NS

    def zb(i, carry):
        pltpu.sync_copy(z16_v, acc_sh.at[pl.ds(s * rpt + i * 16, 16)])
        return carry

    lax.fori_loop(0, rpt // 16, zb, 0)
    plsc.subcore_barrier()

    lo = c * HALF
    iota16 = lax.iota(jnp.int32, 16)
    ept = EP // NS

    def body(i, carry):
        base = s * ept + i * CH
        pltpu.sync_copy(src_hbm.at[pl.ds(base, CH)], src_v)
        pltpu.sync_copy(dst_hbm.at[pl.ds(base, CH)], dst_v)
        for j in range(CH // L):
            sl = pl.ds(j * L, L)
            sv = src_v[sl]
            dv = dst_v[sl]
            inr = (dv >= lo) & (dv < lo + HALF)
            src_v[sl] = jnp.where(inr, sv, N + (j % 8) * L + iota16)
            dst_v[sl] = jnp.where(inr, dv - lo, HALF + (j % 4) * L + iota16)
        pltpu.sync_copy(xws_hbm.at[src_v], rows_v)
        pltpu.sync_copy(rows_v, acc_sh.at[dst_v], add=True)
        return carry

    lax.fori_loop(0, ept // CH, body, 0)
    plsc.subcore_barrier()

    wpt = HALF // NS
    pltpu.sync_copy(
        acc_sh.at[pl.ds(s * wpt, wpt)],
        out_hbm.at[pl.ds(c * HALF + s * wpt, wpt)],
    )


@functools.cache
def _sc_kernels():
    mesh = plsc.VectorSubcoreMesh(
        core_axis_name="c", subcore_axis_name="s", num_cores=NC, num_subcores=NS
    )
    deg_kernel = pl.kernel(
        _deg_body,
        out_type=(
            jax.ShapeDtypeStruct((NP,), _f32),
            jax.ShapeDtypeStruct((NP,), _f32),
        ),
        mesh=mesh,
        scratch_types=[
            pltpu.VMEM((CH,), jnp.int32),
            pltpu.VMEM((CH,), _f32),
            pltpu.VMEM((NP // NS,), _f32),
            pltpu.VMEM_SHARED((NP,), _f32),
        ],
    )
    agg_kernel = pl.kernel(
        _agg_body,
        out_type=jax.ShapeDtypeStruct((NP, 2, 128), _f32),
        mesh=mesh,
        scratch_types=[
            pltpu.VMEM((CH,), jnp.int32),
            pltpu.VMEM((CH,), jnp.int32),
            pltpu.VMEM((CH, 2, 128), _f32),
            pltpu.VMEM((16, 2, 128), _f32),
            pltpu.VMEM_SHARED((ACC_ROWS, 2, 128), _f32),
        ],
    )
    return deg_kernel, agg_kernel


def _gcn_mm_body(x_ref, w_ref, dega_ref, degb_ref, xw_ref, xws_ref):
    xw = jnp.dot(x_ref[...], w_ref[...], preferred_element_type=_f32)
    deg = dega_ref[...] + degb_ref[...] + 1.0
    dinv = lax.rsqrt(deg)
    xw_ref[...] = xw
    xws_ref[...] = xw * dinv


def _mlp1_body(agg_ref, xw_ref, dega_ref, degb_ref, bg_ref, w1_ref, b1_ref, out_ref):
    deg = dega_ref[...] + degb_ref[...] + 1.0
    dinv = lax.rsqrt(deg)
    h = dinv * agg_ref[...] + (dinv * dinv) * xw_ref[...] + bg_ref[...]
    m1 = jnp.dot(h, w1_ref[...], preferred_element_type=_f32) + b1_ref[...]
    out_ref[...] = jnp.maximum(m1, 0.0)


def _mlp2_body(z_ref, w2_ref, b2_ref, out_ref):
    out_ref[...] = (
        jnp.dot(z_ref[...], w2_ref[...], preferred_element_type=_f32) + b2_ref[...]
    )


def kernel(x, edge_index, W_gcn, b_gcn, W1, b1, W2, b2):
    src = edge_index[0].astype(jnp.int32)
    dst = edge_index[1].astype(jnp.int32)
    pad_e = EP - E
    pad_rows = N + (jnp.arange(pad_e, dtype=jnp.int32) % (NP - N))
    srcp = jnp.concatenate([src, pad_rows])
    dstp = jnp.concatenate([dst, pad_rows])
    xp = jnp.pad(x, ((0, NP - N), (0, 0)))

    deg_kernel, agg_kernel = _sc_kernels()
    dega, degb = deg_kernel(dstp)
    dega2 = dega.reshape(NP, 1)
    degb2 = degb.reshape(NP, 1)

    xw, xws = pl.pallas_call(
        _gcn_mm_body,
        out_shape=(
            jax.ShapeDtypeStruct((NP, T), _f32),
            jax.ShapeDtypeStruct((NP, T), _f32),
        ),
    )(xp, W_gcn, dega2, degb2)

    agg = agg_kernel(xws.reshape(NP, 2, 128), srcp, dstp).reshape(NP, T)

    m1 = pl.pallas_call(
        _mlp1_body,
        out_shape=jax.ShapeDtypeStruct((NP, H), _f32),
    )(agg, xw, dega2, degb2, b_gcn.reshape(1, T), W1, b1.reshape(1, H))

    z = m1[:N].reshape(N // H, H * H)
    kpad = (-(H * H)) % 128
    zp = jnp.pad(z, ((0, 0), (0, kpad)))
    w2p = jnp.pad(W2, ((0, kpad), (0, 0)))

    y = pl.pallas_call(
        _mlp2_body,
        out_shape=jax.ShapeDtypeStruct((N // H, H * H), _f32),
    )(zp, w2p, b2.reshape(1, H * H))

    return y.reshape(-1)


# speedup vs baseline: 1.1874x; 1.1874x over previous
import functools

import jax
import jax.numpy as jnp
from jax import lax
from jax.experimental import pallas as pl
from jax.experimental.pallas import tpu as pltpu
from jax.experimental.pallas import tpu_sc as plsc

N = 10000
E = 160000
T = 256
H = 50

NC = 2
NS = 16
L = 16

NP = 10240
EP = 163840
HALF = NP // NC
TRASH = 256
ACC_ROWS = HALF + TRASH
ZROW = NP - 1
CH = 128
ACH = 64
SEG = 1024

_f32 = jnp.float32


def _deg_body(dst_hbm, dega_hbm, degb_hbm, idx_v, ones_v, zslice_v, deg_sh):
    c = lax.axis_index("c")
    s = lax.axis_index("s")
    nslice = NP // NS

    for j in range(CH // L):
        ones_v[pl.ds(j * L, L)] = jnp.ones((L,), _f32)

    def zbody(i, carry):
        zslice_v[pl.ds(i * L, L)] = jnp.zeros((L,), _f32)
        return carry

    lax.fori_loop(0, nslice // L, zbody, 0)
    pltpu.sync_copy(zslice_v, deg_sh.at[pl.ds(s * nslice, nslice)])
    plsc.subcore_barrier()

    ept = EP // (NC * NS)

    def body(i, carry):
        base = (c * NS + s) * ept + i * CH
        pltpu.sync_copy(dst_hbm.at[pl.ds(base, CH)], idx_v)
        pltpu.sync_copy(ones_v, deg_sh.at[idx_v], add=True)
        return carry

    lax.fori_loop(0, ept // CH, body, 0)
    plsc.subcore_barrier()

    @pl.when(c == 0)
    def _():
        pltpu.sync_copy(
            deg_sh.at[pl.ds(s * nslice, nslice)],
            dega_hbm.at[pl.ds(s * nslice, nslice)],
        )

    @pl.when(c == 1)
    def _():
        pltpu.sync_copy(
            deg_sh.at[pl.ds(s * nslice, nslice)],
            degb_hbm.at[pl.ds(s * nslice, nslice)],
        )


def _agg_body(xws_hbm, src_hbm, dst_hbm, out_hbm,
              sstage, dstage, dst_v, rows0, rows1, z16_v, semA, semB, acc_sh):
    c = lax.axis_index("c")
    s = lax.axis_index("s")

    for i in range(16):
        for h in range(2):
            for j in range(128 // L):
                z16_v[i, h, pl.ds(j * L, L)] = jnp.zeros((L,), _f32)

    rpt = ACC_ROWS // NS

    def zb(i, carry):
        pltpu.sync_copy(z16_v, acc_sh.at[pl.ds(s * rpt + i * 16, 16)])
        return carry

    lax.fori_loop(0, rpt // 16, zb, 0)
    plsc.subcore_barrier()

    lo = c * HALF
    iota16 = lax.iota(jnp.int32, 16)
    ept = EP // NS

    def prep_segment(g):
        base = s * ept + g * SEG
        pltpu.sync_copy(src_hbm.at[pl.ds(base, SEG)], sstage)
        pltpu.sync_copy(dst_hbm.at[pl.ds(base, SEG)], dstage)

        def tb(t, carry):
            sl = pl.ds(t * L, L)
            sv = sstage[sl]
            dv = dstage[sl]
            inr = (dv >= lo) & (dv < lo + HALF)
            sent = (t % 8) * L + iota16
            sstage[sl] = jnp.where(inr, sv, N + sent)
            dstage[sl] = jnp.where(inr, dv - lo, HALF + sent)
            return carry

        lax.fori_loop(0, SEG // L, tb, 0)

    def gather(k, rows, sem):
        pltpu.async_copy(xws_hbm.at[sstage.at[pl.ds(k * ACH, ACH)]], rows, sem)

    def drain(rows, sem):
        pltpu.make_async_copy(xws_hbm.at[sstage.at[pl.ds(0, ACH)]], rows, sem).wait()

    def scatter(k, rows):
        for j in range(ACH // L):
            dst_v[pl.ds(j * L, L)] = dstage[pl.ds(k * ACH + j * L, L)]
        pltpu.sync_copy(rows, acc_sh.at[dst_v], add=True)

    cps = SEG // ACH

    def seg_body(g, carry):
        prep_segment(g)
        gather(0, rows0, semA)

        def outer(g2, carry2):
            k0 = 2 * g2
            gather(k0 + 1, rows1, semB)
            drain(rows0, semA)
            scatter(k0, rows0)

            @pl.when(g2 + 1 < cps // 2)
            def _():
                gather(k0 + 2, rows0, semA)

            drain(rows1, semB)
            scatter(k0 + 1, rows1)
            return carry2

        lax.fori_loop(0, cps // 2, outer, 0)
        return carry

    lax.fori_loop(0, ept // SEG, seg_body, 0)
    plsc.subcore_barrier()

    wpt = HALF // NS
    pltpu.sync_copy(
        acc_sh.at[pl.ds(s * wpt, wpt)],
        out_hbm.at[pl.ds(c * HALF + s * wpt, wpt)],
    )


@functools.cache
def _sc_kernels():
    mesh = plsc.VectorSubcoreMesh(
        core_axis_name="c", subcore_axis_name="s", num_cores=NC, num_subcores=NS
    )
    deg_kernel = pl.kernel(
        _deg_body,
        out_type=(
            jax.ShapeDtypeStruct((NP,), _f32),
            jax.ShapeDtypeStruct((NP,), _f32),
        ),
        mesh=mesh,
        scratch_types=[
            pltpu.VMEM((CH,), jnp.int32),
            pltpu.VMEM((CH,), _f32),
            pltpu.VMEM((NP // NS,), _f32),
            pltpu.VMEM_SHARED((NP,), _f32),
        ],
    )
    agg_kernel = pl.kernel(
        _agg_body,
        out_type=jax.ShapeDtypeStruct((NP, 2, 128), _f32),
        mesh=mesh,
        scratch_types=[
            pltpu.VMEM((SEG,), jnp.int32),
            pltpu.VMEM((SEG,), jnp.int32),
            pltpu.VMEM((ACH,), jnp.int32),
            pltpu.VMEM((ACH, 2, 128), _f32),
            pltpu.VMEM((ACH, 2, 128), _f32),
            pltpu.VMEM((16, 2, 128), _f32),
            pltpu.SemaphoreType.DMA,
            pltpu.SemaphoreType.DMA,
            pltpu.VMEM_SHARED((ACC_ROWS, 2, 128), _f32),
        ],
    )
    return deg_kernel, agg_kernel


def _gcn_mm_body(x_ref, w_ref, dega_ref, degb_ref, xw_ref, xws_ref):
    xw = jnp.dot(x_ref[...], w_ref[...], preferred_element_type=_f32)
    deg = dega_ref[...] + degb_ref[...] + 1.0
    dinv = lax.rsqrt(deg)
    xw_ref[...] = xw
    xws_ref[...] = xw * dinv


def _mlp1_body(agg_ref, xw_ref, dega_ref, degb_ref, bg_ref, w1_ref, b1_ref, out_ref):
    deg = dega_ref[...] + degb_ref[...] + 1.0
    dinv = lax.rsqrt(deg)
    h = dinv * agg_ref[...] + (dinv * dinv) * xw_ref[...] + bg_ref[...]
    m1 = jnp.dot(h, w1_ref[...], preferred_element_type=_f32) + b1_ref[...]
    out_ref[...] = jnp.maximum(m1, 0.0)


def _mlp2_body(z_ref, w2_ref, b2_ref, out_ref):
    out_ref[...] = (
        jnp.dot(z_ref[...], w2_ref[...], preferred_element_type=_f32) + b2_ref[...]
    )


def kernel(x, edge_index, W_gcn, b_gcn, W1, b1, W2, b2):
    src = edge_index[0].astype(jnp.int32)
    dst = edge_index[1].astype(jnp.int32)
    pad_e = EP - E
    pad_rows = N + (jnp.arange(pad_e, dtype=jnp.int32) % (NP - N))
    srcp = jnp.concatenate([src, pad_rows])
    dstp = jnp.concatenate([dst, pad_rows])
    xp = jnp.pad(x, ((0, NP - N), (0, 0)))

    deg_kernel, agg_kernel = _sc_kernels()
    dega, degb = deg_kernel(dstp)
    dega2 = dega.reshape(NP, 1)
    degb2 = degb.reshape(NP, 1)

    xw, xws = pl.pallas_call(
        _gcn_mm_body,
        out_shape=(
            jax.ShapeDtypeStruct((NP, T), _f32),
            jax.ShapeDtypeStruct((NP, T), _f32),
        ),
    )(xp, W_gcn, dega2, degb2)

    agg = agg_kernel(xws.reshape(NP, 2, 128), srcp, dstp).reshape(NP, T)

    m1 = pl.pallas_call(
        _mlp1_body,
        out_shape=jax.ShapeDtypeStruct((NP, H), _f32),
    )(agg, xw, dega2, degb2, b_gcn.reshape(1, T), W1, b1.reshape(1, H))

    z = m1[:N].reshape(N // H, H * H)
    kpad = (-(H * H)) % 128
    zp = jnp.pad(z, ((0, 0), (0, kpad)))
    w2p = jnp.pad(W2, ((0, kpad), (0, 0)))

    y = pl.pallas_call(
        _mlp2_body,
        out_shape=jax.ShapeDtypeStruct((N // H, H * H), _f32),
    )(zp, w2p, b2.reshape(1, H * H))

    return y.reshape(-1)


# speedup vs baseline: 1.6317x; 1.3742x over previous
import functools

import jax
import jax.numpy as jnp
from jax import lax
from jax.experimental import pallas as pl
from jax.experimental.pallas import tpu as pltpu
from jax.experimental.pallas import tpu_sc as plsc

N = 10000
E = 160000
T = 256
H = 50

NC = 2
NS = 16
L = 16

NP = 10240
EP = 163840
HALF = NP // NC
TRASH = 256
ACC_ROWS = HALF + TRASH
ZROW = NP - 1
CH = 128
ACH = 64
SEG = 1024

_f32 = jnp.float32


def _deg_body(dst_hbm, dega_hbm, degb_hbm, idx_v, ones_v, zslice_v, deg_sh):
    c = lax.axis_index("c")
    s = lax.axis_index("s")
    nslice = NP // NS

    for j in range(CH // L):
        ones_v[pl.ds(j * L, L)] = jnp.ones((L,), _f32)

    def zbody(i, carry):
        zslice_v[pl.ds(i * L, L)] = jnp.zeros((L,), _f32)
        return carry

    lax.fori_loop(0, nslice // L, zbody, 0)
    pltpu.sync_copy(zslice_v, deg_sh.at[pl.ds(s * nslice, nslice)])
    plsc.subcore_barrier()

    ept = EP // (NC * NS)

    def body(i, carry):
        base = (c * NS + s) * ept + i * CH
        pltpu.sync_copy(dst_hbm.at[pl.ds(base, CH)], idx_v)
        pltpu.sync_copy(ones_v, deg_sh.at[idx_v], add=True)
        return carry

    lax.fori_loop(0, ept // CH, body, 0)
    plsc.subcore_barrier()

    @pl.when(c == 0)
    def _():
        pltpu.sync_copy(
            deg_sh.at[pl.ds(s * nslice, nslice)],
            dega_hbm.at[pl.ds(s * nslice, nslice)],
        )

    @pl.when(c == 1)
    def _():
        pltpu.sync_copy(
            deg_sh.at[pl.ds(s * nslice, nslice)],
            degb_hbm.at[pl.ds(s * nslice, nslice)],
        )


def _agg_body(xws_hbm, src_hbm, dst_hbm, out_hbm,
              sstage, dstage, dst_v, rows0, rows1, z16_v, semA, semB, acc_sh):
    c = lax.axis_index("c")
    s = lax.axis_index("s")

    for i in range(16):
        for h in range(2):
            for j in range(128 // L):
                z16_v[i, h, pl.ds(j * L, L)] = jnp.zeros((L,), _f32)

    rpt = ACC_ROWS // NS

    def zb(i, carry):
        pltpu.sync_copy(z16_v, acc_sh.at[pl.ds(s * rpt + i * 16, 16)])
        return carry

    lax.fori_loop(0, rpt // 16, zb, 0)
    plsc.subcore_barrier()

    lo = c * HALF
    iota16 = lax.iota(jnp.int32, 16)
    ept = EP // NS

    def prep_segment(g):
        base = s * ept + g * SEG
        pltpu.sync_copy(src_hbm.at[pl.ds(base, SEG)], sstage.at[pl.ds(0, SEG)])
        pltpu.sync_copy(dst_hbm.at[pl.ds(base, SEG)], dstage.at[pl.ds(0, SEG)])

        def tb(t, cnt):
            sl = pl.ds(t * L, L)
            sv = sstage[sl]
            dv = dstage[sl]
            m = (dv >= lo) & (dv < lo + HALF)
            plsc.store_compressed(sstage.at[pl.ds(cnt, L)], sv, mask=m)
            plsc.store_compressed(dstage.at[pl.ds(cnt, L)], dv - lo, mask=m)
            return cnt + plsc.all_reduce_population_count(m)[0]

        cnt = lax.fori_loop(0, SEG // L, tb, jnp.int32(0))

        def fb(k, carry):
            off = cnt + k * L
            sent = (k % 8) * L + iota16
            sstage[pl.ds(off, L)] = N + sent
            dstage[pl.ds(off, L)] = HALF + sent
            return carry

        lax.fori_loop(0, 2 * (ACH // L), fb, 0)
        return cnt

    def gather(k, rows, sem):
        pltpu.async_copy(xws_hbm.at[sstage.at[pl.ds(k * ACH, ACH)]], rows, sem)

    def drain(rows, sem):
        pltpu.make_async_copy(xws_hbm.at[sstage.at[pl.ds(0, ACH)]], rows, sem).wait()

    def scatter(k, rows):
        for j in range(ACH // L):
            dst_v[pl.ds(j * L, L)] = dstage[pl.ds(k * ACH + j * L, L)]
        pltpu.sync_copy(rows, acc_sh.at[dst_v], add=True)

    def seg_body(g, carry):
        cnt = prep_segment(g)
        nch = (cnt + ACH - 1) // ACH
        npairs = (nch + 1) // 2

        @pl.when(npairs > 0)
        def _():
            gather(0, rows0, semA)

        def outer(g2, carry2):
            k0 = 2 * g2
            gather(k0 + 1, rows1, semB)
            drain(rows0, semA)
            scatter(k0, rows0)

            @pl.when(g2 + 1 < npairs)
            def _():
                gather(k0 + 2, rows0, semA)

            drain(rows1, semB)
            scatter(k0 + 1, rows1)
            return carry2

        lax.fori_loop(0, npairs, outer, 0)
        return carry

    lax.fori_loop(0, ept // SEG, seg_body, 0)
    plsc.subcore_barrier()

    wpt = HALF // NS
    pltpu.sync_copy(
        acc_sh.at[pl.ds(s * wpt, wpt)],
        out_hbm.at[pl.ds(c * HALF + s * wpt, wpt)],
    )


@functools.cache
def _sc_kernels():
    mesh = plsc.VectorSubcoreMesh(
        core_axis_name="c", subcore_axis_name="s", num_cores=NC, num_subcores=NS
    )
    deg_kernel = pl.kernel(
        _deg_body,
        out_type=(
            jax.ShapeDtypeStruct((NP,), _f32),
            jax.ShapeDtypeStruct((NP,), _f32),
        ),
        mesh=mesh,
        scratch_types=[
            pltpu.VMEM((CH,), jnp.int32),
            pltpu.VMEM((CH,), _f32),
            pltpu.VMEM((NP // NS,), _f32),
            pltpu.VMEM_SHARED((NP,), _f32),
        ],
    )
    agg_kernel = pl.kernel(
        _agg_body,
        out_type=jax.ShapeDtypeStruct((NP, 2, 128), _f32),
        mesh=mesh,
        compiler_params=pltpu.CompilerParams(needs_layout_passes=False),
        scratch_types=[
            pltpu.VMEM((SEG + 2 * ACH,), jnp.int32),
            pltpu.VMEM((SEG + 2 * ACH,), jnp.int32),
            pltpu.VMEM((ACH,), jnp.int32),
            pltpu.VMEM((ACH, 2, 128), _f32),
            pltpu.VMEM((ACH, 2, 128), _f32),
            pltpu.VMEM((16, 2, 128), _f32),
            pltpu.SemaphoreType.DMA,
            pltpu.SemaphoreType.DMA,
            pltpu.VMEM_SHARED((ACC_ROWS, 2, 128), _f32),
        ],
    )
    return deg_kernel, agg_kernel


def _gcn_mm_body(x_ref, w_ref, dega_ref, degb_ref, xw_ref, xws_ref):
    xw = jnp.dot(x_ref[...], w_ref[...], preferred_element_type=_f32)
    deg = dega_ref[...] + degb_ref[...] + 1.0
    dinv = lax.rsqrt(deg)
    xw_ref[...] = xw
    xws_ref[...] = xw * dinv


def _mlp1_body(agg_ref, xw_ref, dega_ref, degb_ref, bg_ref, w1_ref, b1_ref, out_ref):
    deg = dega_ref[...] + degb_ref[...] + 1.0
    dinv = lax.rsqrt(deg)
    h = dinv * agg_ref[...] + (dinv * dinv) * xw_ref[...] + bg_ref[...]
    m1 = jnp.dot(h, w1_ref[...], preferred_element_type=_f32) + b1_ref[...]
    out_ref[...] = jnp.maximum(m1, 0.0)


def _mlp2_body(z_ref, w2_ref, b2_ref, out_ref):
    out_ref[...] = (
        jnp.dot(z_ref[...], w2_ref[...], preferred_element_type=_f32) + b2_ref[...]
    )


def kernel(x, edge_index, W_gcn, b_gcn, W1, b1, W2, b2):
    src = edge_index[0].astype(jnp.int32)
    dst = edge_index[1].astype(jnp.int32)
    pad_e = EP - E
    pad_rows = N + (jnp.arange(pad_e, dtype=jnp.int32) % (NP - N))
    srcp = jnp.concatenate([src, pad_rows])
    dstp = jnp.concatenate([dst, pad_rows])
    xp = jnp.pad(x, ((0, NP - N), (0, 0)))

    deg_kernel, agg_kernel = _sc_kernels()
    dega, degb = deg_kernel(dstp)
    dega2 = dega.reshape(NP, 1)
    degb2 = degb.reshape(NP, 1)

    xw, xws = pl.pallas_call(
        _gcn_mm_body,
        out_shape=(
            jax.ShapeDtypeStruct((NP, T), _f32),
            jax.ShapeDtypeStruct((NP, T), _f32),
        ),
    )(xp, W_gcn, dega2, degb2)

    agg = agg_kernel(xws.reshape(NP, 2, 128), srcp, dstp).reshape(NP, T)

    m1 = pl.pallas_call(
        _mlp1_body,
        out_shape=jax.ShapeDtypeStruct((NP, H), _f32),
    )(agg, xw, dega2, degb2, b_gcn.reshape(1, T), W1, b1.reshape(1, H))

    z = m1[:N].reshape(N // H, H * H)
    kpad = (-(H * H)) % 128
    zp = jnp.pad(z, ((0, 0), (0, kpad)))
    w2p = jnp.pad(W2, ((0, kpad), (0, 0)))

    y = pl.pallas_call(
        _mlp2_body,
        out_shape=jax.ShapeDtypeStruct((N // H, H * H), _f32),
    )(zp, w2p, b2.reshape(1, H * H))

    return y.reshape(-1)


# speedup vs baseline: 1.7146x; 1.0508x over previous
import functools

import jax
import jax.numpy as jnp
from jax import lax
from jax.experimental import pallas as pl
from jax.experimental.pallas import tpu as pltpu
from jax.experimental.pallas import tpu_sc as plsc

N = 10000
E = 160000
T = 256
H = 50

NC = 2
NS = 16
L = 16

NP = 10240
EP = 163840
HALF = NP // NC
TRASH = 256
ACC_ROWS = HALF + TRASH
ZROW = NP - 1
CH = 128
ACH = 64
SEG = 1024

_f32 = jnp.float32


def _deg_body(dst_hbm, dega_hbm, degb_hbm, idx_v, ones_v, zslice_v, deg_sh):
    c = lax.axis_index("c")
    s = lax.axis_index("s")
    nslice = NP // NS

    for j in range(CH // L):
        ones_v[pl.ds(j * L, L)] = jnp.ones((L,), _f32)

    def zbody(i, carry):
        zslice_v[pl.ds(i * L, L)] = jnp.zeros((L,), _f32)
        return carry

    lax.fori_loop(0, nslice // L, zbody, 0)
    pltpu.sync_copy(zslice_v, deg_sh.at[pl.ds(s * nslice, nslice)])
    plsc.subcore_barrier()

    ept = EP // (NC * NS)

    def body(i, carry):
        base = (c * NS + s) * ept + i * CH
        pltpu.sync_copy(dst_hbm.at[pl.ds(base, CH)], idx_v)
        pltpu.sync_copy(ones_v, deg_sh.at[idx_v], add=True)
        return carry

    lax.fori_loop(0, ept // CH, body, 0)
    plsc.subcore_barrier()

    @pl.when(c == 0)
    def _():
        pltpu.sync_copy(
            deg_sh.at[pl.ds(s * nslice, nslice)],
            dega_hbm.at[pl.ds(s * nslice, nslice)],
        )

    @pl.when(c == 1)
    def _():
        pltpu.sync_copy(
            deg_sh.at[pl.ds(s * nslice, nslice)],
            degb_hbm.at[pl.ds(s * nslice, nslice)],
        )


def _agg_body(xws_hbm, src_hbm, dst_hbm, out_hbm,
              sstage, dstage, dst_v, rows0, rows1, z16_v, semA, semB, acc_sh):
    c = lax.axis_index("c")
    s = lax.axis_index("s")

    for i in range(16):
        for h in range(2):
            for j in range(128 // L):
                z16_v[i, h, pl.ds(j * L, L)] = jnp.zeros((L,), _f32)

    rpt = ACC_ROWS // NS

    def zb(i, carry):
        pltpu.sync_copy(z16_v, acc_sh.at[pl.ds(s * rpt + i * 16, 16)])
        return carry

    lax.fori_loop(0, rpt // 16, zb, 0)
    plsc.subcore_barrier()

    lo = c * HALF
    iota16 = lax.iota(jnp.int32, 16)
    ept = EP // NS

    def prep_segment(g):
        base = s * ept + g * SEG
        pltpu.sync_copy(src_hbm.at[pl.ds(base, SEG)], sstage.at[pl.ds(0, SEG)])
        pltpu.sync_copy(dst_hbm.at[pl.ds(base, SEG)], dstage.at[pl.ds(0, SEG)])

        def tb(t, cnt):
            sl = pl.ds(t * L, L)
            sv = sstage[sl]
            dv = dstage[sl]
            m = (dv >= lo) & (dv < lo + HALF)
            plsc.store_compressed(sstage.at[pl.ds(cnt, L)], sv, mask=m)
            plsc.store_compressed(dstage.at[pl.ds(cnt, L)], dv - lo, mask=m)
            return cnt + plsc.all_reduce_population_count(m)[0]

        cnt = lax.fori_loop(0, SEG // L, tb, jnp.int32(0))

        def fb(k, carry):
            off = cnt + k * L
            sent = (k % 8) * L + iota16
            sstage[pl.ds(off, L)] = N + sent
            dstage[pl.ds(off, L)] = HALF + sent
            return carry

        lax.fori_loop(0, 2 * (ACH // L), fb, 0)
        return cnt

    def gather(k, rows, sem):
        pltpu.async_copy(xws_hbm.at[sstage.at[pl.ds(k * ACH, ACH)]], rows, sem)

    def drain(rows, sem):
        pltpu.make_async_copy(xws_hbm.at[sstage.at[pl.ds(0, ACH)]], rows, sem).wait()

    def scatter(k, rows):
        for j in range(ACH // L):
            dst_v[pl.ds(j * L, L)] = dstage[pl.ds(k * ACH + j * L, L)]
        pltpu.sync_copy(rows, acc_sh.at[dst_v], add=True)

    def seg_body(g, carry):
        cnt = prep_segment(g)
        nch = (cnt + ACH - 1) // ACH
        npairs = (nch + 1) // 2

        @pl.when(npairs > 0)
        def _():
            gather(0, rows0, semA)

        def outer(g2, carry2):
            k0 = 2 * g2
            gather(k0 + 1, rows1, semB)
            drain(rows0, semA)
            scatter(k0, rows0)

            @pl.when(g2 + 1 < npairs)
            def _():
                gather(k0 + 2, rows0, semA)

            drain(rows1, semB)
            scatter(k0 + 1, rows1)
            return carry2

        lax.fori_loop(0, npairs, outer, 0)
        return carry

    lax.fori_loop(0, ept // SEG, seg_body, 0)
    plsc.subcore_barrier()

    wpt = HALF // NS
    pltpu.sync_copy(
        acc_sh.at[pl.ds(s * wpt, wpt)],
        out_hbm.at[pl.ds(c * HALF + s * wpt, wpt)],
    )


@functools.cache
def _sc_kernels():
    mesh = plsc.VectorSubcoreMesh(
        core_axis_name="c", subcore_axis_name="s", num_cores=NC, num_subcores=NS
    )
    deg_kernel = pl.kernel(
        _deg_body,
        out_type=(
            jax.ShapeDtypeStruct((NP,), _f32),
            jax.ShapeDtypeStruct((NP,), _f32),
        ),
        mesh=mesh,
        scratch_types=[
            pltpu.VMEM((CH,), jnp.int32),
            pltpu.VMEM((CH,), _f32),
            pltpu.VMEM((NP // NS,), _f32),
            pltpu.VMEM_SHARED((NP,), _f32),
        ],
    )
    agg_kernel = pl.kernel(
        _agg_body,
        out_type=jax.ShapeDtypeStruct((NP, 2, 128), _f32),
        mesh=mesh,
        compiler_params=pltpu.CompilerParams(needs_layout_passes=False),
        scratch_types=[
            pltpu.VMEM((SEG + 2 * ACH,), jnp.int32),
            pltpu.VMEM((SEG + 2 * ACH,), jnp.int32),
            pltpu.VMEM((ACH,), jnp.int32),
            pltpu.VMEM((ACH, 2, 128), _f32),
            pltpu.VMEM((ACH, 2, 128), _f32),
            pltpu.VMEM((16, 2, 128), _f32),
            pltpu.SemaphoreType.DMA,
            pltpu.SemaphoreType.DMA,
            pltpu.VMEM_SHARED((ACC_ROWS, 2, 128), _f32),
        ],
    )
    return deg_kernel, agg_kernel


def _gcn_mm_body(x_ref, w_ref, dega_ref, degb_ref, xw_ref, xws_ref):
    xw = jnp.dot(x_ref[...], w_ref[...], preferred_element_type=_f32)
    deg = dega_ref[...] + degb_ref[...] + 1.0
    dinv = lax.rsqrt(deg)
    xw_ref[...] = xw
    xws_ref[...] = xw * dinv


def _mlp1_body(agg_ref, xw_ref, dega_ref, degb_ref, bg_ref, w1_ref, b1_ref, out_ref):
    deg = dega_ref[...] + degb_ref[...] + 1.0
    dinv = lax.rsqrt(deg)
    h = dinv * agg_ref[...] + (dinv * dinv) * xw_ref[...] + bg_ref[...]
    m1 = jnp.dot(h, w1_ref[...], preferred_element_type=_f32) + b1_ref[...]
    out_ref[...] = jnp.maximum(m1, 0.0)


def _mlp2_body(z_ref, w2_ref, b2_ref, out_ref):
    out_ref[...] = (
        jnp.dot(z_ref[...], w2_ref[...], preferred_element_type=_f32) + b2_ref[...]
    )


def kernel(x, edge_index, W_gcn, b_gcn, W1, b1, W2, b2):
    src = edge_index[0].astype(jnp.int32)
    dst = edge_index[1].astype(jnp.int32)
    pad_e = EP - E
    pad_rows = N + (jnp.arange(pad_e, dtype=jnp.int32) % (NP - N))
    srcp = jnp.concatenate([src, pad_rows])
    dstp = jnp.concatenate([dst, pad_rows])
    xp = jnp.pad(x, ((0, NP - N), (0, 0)))

    deg_kernel, agg_kernel = _sc_kernels()
    dega, degb = deg_kernel(dstp)
    dega2 = dega.reshape(NP, 1)
    degb2 = degb.reshape(NP, 1)

    xw, xws = pl.pallas_call(
        _gcn_mm_body,
        out_shape=(
            jax.ShapeDtypeStruct((NP, T), _f32),
            jax.ShapeDtypeStruct((NP, T), _f32),
        ),
    )(xp, W_gcn, dega2, degb2)

    agg = agg_kernel(xws.reshape(NP, 2, 128), srcp, dstp).reshape(NP, T)

    m1 = pl.pallas_call(
        _mlp1_body,
        out_shape=jax.ShapeDtypeStruct((NP, H), _f32),
    )(agg, xw, dega2, degb2, b_gcn.reshape(1, T), W1, b1.reshape(1, H))

    z = m1[:N].reshape(N // H, H * H)

    y = pl.pallas_call(
        _mlp2_body,
        out_shape=jax.ShapeDtypeStruct((N // H, H * H), _f32),
    )(z, W2, b2.reshape(1, H * H))

    return y.reshape(-1)


# speedup vs baseline: 1.7897x; 1.0438x over previous
import functools

import jax
import jax.numpy as jnp
from jax import lax
from jax.experimental import pallas as pl
from jax.experimental.pallas import tpu as pltpu
from jax.experimental.pallas import tpu_sc as plsc

N = 10000
E = 160000
T = 256
H = 50

NC = 2
NS = 16
L = 16

NP = 10240
EP = 163840
HALF = NP // NC
TRASH = 256
ACC_ROWS = HALF + TRASH
ZROW = NP - 1
CH = 128
ACH = 64
SEG = 1024

_f32 = jnp.float32


def _deg_body(ei_hbm, dega_hbm, degb_hbm, idx_v, ones_v, zslice_v, deg_sh):
    c = lax.axis_index("c")
    s = lax.axis_index("s")
    nslice = NP // NS

    for j in range(CH // L):
        ones_v[pl.ds(j * L, L)] = jnp.ones((L,), _f32)

    def zbody(i, carry):
        zslice_v[pl.ds(i * L, L)] = jnp.zeros((L,), _f32)
        return carry

    lax.fori_loop(0, nslice // L, zbody, 0)
    pltpu.sync_copy(zslice_v, deg_sh.at[pl.ds(s * nslice, nslice)])
    plsc.subcore_barrier()

    nch = E // CH
    tid = c * NS + s

    def body(i, carry):
        ci = tid + i * NC * NS

        @pl.when(ci < nch)
        def _():
            pltpu.sync_copy(ei_hbm.at[1, pl.ds(ci * CH, CH)], idx_v)
            pltpu.sync_copy(ones_v, deg_sh.at[idx_v], add=True)

        return carry

    lax.fori_loop(0, (nch + NC * NS - 1) // (NC * NS), body, 0)
    plsc.subcore_barrier()

    @pl.when(c == 0)
    def _():
        pltpu.sync_copy(
            deg_sh.at[pl.ds(s * nslice, nslice)],
            dega_hbm.at[pl.ds(s * nslice, nslice)],
        )

    @pl.when(c == 1)
    def _():
        pltpu.sync_copy(
            deg_sh.at[pl.ds(s * nslice, nslice)],
            degb_hbm.at[pl.ds(s * nslice, nslice)],
        )


def _agg_body(xws_hbm, ei_hbm, out_hbm,
              sstage, dstage, dst_v, rows0, rows1, z16_v, semA, semB, acc_sh):
    c = lax.axis_index("c")
    s = lax.axis_index("s")

    for i in range(16):
        for h in range(2):
            for j in range(128 // L):
                z16_v[i, h, pl.ds(j * L, L)] = jnp.zeros((L,), _f32)

    rpt = ACC_ROWS // NS

    def zb(i, carry):
        pltpu.sync_copy(z16_v, acc_sh.at[pl.ds(s * rpt + i * 16, 16)])
        return carry

    lax.fori_loop(0, rpt // 16, zb, 0)
    plsc.subcore_barrier()

    lo = c * HALF
    iota16 = lax.iota(jnp.int32, 16)

    def prep_segment(base, seglen):
        pltpu.sync_copy(ei_hbm.at[0, pl.ds(base, seglen)], sstage.at[pl.ds(0, seglen)])
        pltpu.sync_copy(ei_hbm.at[1, pl.ds(base, seglen)], dstage.at[pl.ds(0, seglen)])

        def tb(t, cnt):
            sl = pl.ds(t * L, L)
            sv = sstage[sl]
            dv = dstage[sl]
            m = (dv >= lo) & (dv < lo + HALF)
            plsc.store_compressed(sstage.at[pl.ds(cnt, L)], sv, mask=m)
            plsc.store_compressed(dstage.at[pl.ds(cnt, L)], dv - lo, mask=m)
            return cnt + plsc.all_reduce_population_count(m)[0]

        cnt = lax.fori_loop(0, seglen // L, tb, jnp.int32(0))

        def fb(k, carry):
            off = cnt + k * L
            sent = (k % 8) * L + iota16
            sstage[pl.ds(off, L)] = N + sent
            dstage[pl.ds(off, L)] = HALF + sent
            return carry

        lax.fori_loop(0, 2 * (ACH // L), fb, 0)
        return cnt

    def gather(k, rows, sem):
        pltpu.async_copy(xws_hbm.at[sstage.at[pl.ds(k * ACH, ACH)]], rows, sem)

    def drain(rows, sem):
        pltpu.make_async_copy(xws_hbm.at[sstage.at[pl.ds(0, ACH)]], rows, sem).wait()

    def scatter(k, rows):
        for j in range(ACH // L):
            dst_v[pl.ds(j * L, L)] = dstage[pl.ds(k * ACH + j * L, L)]
        pltpu.sync_copy(rows, acc_sh.at[dst_v], add=True)

    def seg_run(base, seglen):
        cnt = prep_segment(base, seglen)
        nch = (cnt + ACH - 1) // ACH
        npairs = (nch + 1) // 2

        @pl.when(npairs > 0)
        def _():
            gather(0, rows0, semA)

        def outer(g2, carry2):
            k0 = 2 * g2
            gather(k0 + 1, rows1, semB)
            drain(rows0, semA)
            scatter(k0, rows0)

            @pl.when(g2 + 1 < npairs)
            def _():
                gather(k0 + 2, rows0, semA)

            drain(rows1, semB)
            scatter(k0 + 1, rows1)
            return carry2

        lax.fori_loop(0, npairs, outer, 0)

    nseg = E // SEG

    def seg_body(i, carry):
        si = s + i * NS

        @pl.when(si < nseg)
        def _():
            seg_run(si * SEG, SEG)

        return carry

    lax.fori_loop(0, (nseg + NS - 1) // NS, seg_body, 0)

    @pl.when(s == nseg % NS)
    def _():
        seg_run(nseg * SEG, E - nseg * SEG)

    plsc.subcore_barrier()

    wpt = HALF // NS
    pltpu.sync_copy(
        acc_sh.at[pl.ds(s * wpt, wpt)],
        out_hbm.at[pl.ds(c * HALF + s * wpt, wpt)],
    )


@functools.cache
def _sc_kernels():
    mesh = plsc.VectorSubcoreMesh(
        core_axis_name="c", subcore_axis_name="s", num_cores=NC, num_subcores=NS
    )
    deg_kernel = pl.kernel(
        _deg_body,
        out_type=(
            jax.ShapeDtypeStruct((NP,), _f32),
            jax.ShapeDtypeStruct((NP,), _f32),
        ),
        mesh=mesh,
        scratch_types=[
            pltpu.VMEM((CH,), jnp.int32),
            pltpu.VMEM((CH,), _f32),
            pltpu.VMEM((NP // NS,), _f32),
            pltpu.VMEM_SHARED((NP,), _f32),
        ],
    )
    agg_kernel = pl.kernel(
        _agg_body,
        out_type=jax.ShapeDtypeStruct((NP, 2, 128), _f32),
        mesh=mesh,
        compiler_params=pltpu.CompilerParams(needs_layout_passes=False),
        scratch_types=[
            pltpu.VMEM((SEG + 2 * ACH,), jnp.int32),
            pltpu.VMEM((SEG + 2 * ACH,), jnp.int32),
            pltpu.VMEM((ACH,), jnp.int32),
            pltpu.VMEM((ACH, 2, 128), _f32),
            pltpu.VMEM((ACH, 2, 128), _f32),
            pltpu.VMEM((16, 2, 128), _f32),
            pltpu.SemaphoreType.DMA,
            pltpu.SemaphoreType.DMA,
            pltpu.VMEM_SHARED((ACC_ROWS, 2, 128), _f32),
        ],
    )
    return deg_kernel, agg_kernel


def _gcn_mm_body(x_ref, w_ref, dega_ref, degb_ref, xw_ref, xws_ref):
    xw = jnp.dot(x_ref[...], w_ref[...], preferred_element_type=_f32)
    deg = dega_ref[...] + degb_ref[...] + 1.0
    dinv = lax.rsqrt(deg)
    zpad = jnp.zeros((NP - N, T), _f32)
    xw_ref[...] = jnp.concatenate([xw, zpad], axis=0)
    xws_ref[...] = jnp.concatenate([xw * dinv[0:N], zpad], axis=0)


def _mlp1_body(agg_ref, xw_ref, dega_ref, degb_ref, bg_ref, w1_ref, b1_ref, out_ref):
    deg = dega_ref[...] + degb_ref[...] + 1.0
    dinv = lax.rsqrt(deg)
    h = dinv * agg_ref[...] + (dinv * dinv) * xw_ref[...] + bg_ref[...]
    m1 = jnp.dot(h, w1_ref[...], preferred_element_type=_f32) + b1_ref[...]
    out_ref[...] = jnp.maximum(m1, 0.0)


def _mlp2_body(z_ref, w2_ref, b2_ref, out_ref):
    out_ref[...] = (
        jnp.dot(z_ref[...], w2_ref[...], preferred_element_type=_f32) + b2_ref[...]
    )


def kernel(x, edge_index, W_gcn, b_gcn, W1, b1, W2, b2):
    ei = edge_index.astype(jnp.int32)

    deg_kernel, agg_kernel = _sc_kernels()
    dega, degb = deg_kernel(ei)
    dega2 = dega.reshape(NP, 1)
    degb2 = degb.reshape(NP, 1)

    xw, xws = pl.pallas_call(
        _gcn_mm_body,
        out_shape=(
            jax.ShapeDtypeStruct((NP, T), _f32),
            jax.ShapeDtypeStruct((NP, T), _f32),
        ),
    )(x, W_gcn, dega2, degb2)

    agg = agg_kernel(xws.reshape(NP, 2, 128), ei).reshape(NP, T)

    m1 = pl.pallas_call(
        _mlp1_body,
        out_shape=jax.ShapeDtypeStruct((NP, H), _f32),
    )(agg, xw, dega2, degb2, b_gcn.reshape(1, T), W1, b1.reshape(1, H))

    z = m1[:N].reshape(N // H, H * H)

    y = pl.pallas_call(
        _mlp2_body,
        out_shape=jax.ShapeDtypeStruct((N // H, H * H), _f32),
    )(z, W2, b2.reshape(1, H * H))

    return y.reshape(-1)


# speedup vs baseline: 1.8594x; 1.0389x over previous
import functools

import jax
import jax.numpy as jnp
from jax import lax
from jax.experimental import pallas as pl
from jax.experimental.pallas import tpu as pltpu
from jax.experimental.pallas import tpu_sc as plsc

N = 10000
E = 160000
T = 256
H = 50

NC = 2
NS = 16
L = 16

NP = 10240
HALF = NP // NC
TRASH = 256
ACC_ROWS = HALF + TRASH
CH = 128
ACH = 64
SEG = 1024

_f32 = jnp.float32


def _deg_body(ei_hbm, dega_hbm, degb_hbm, idxA, idxB, ones_v, zslice_v,
              semA, semB, deg_sh):
    c = lax.axis_index("c")
    s = lax.axis_index("s")
    nslice = NP // NS

    for j in range(CH // L):
        ones_v[pl.ds(j * L, L)] = jnp.ones((L,), _f32)

    def zbody(i, carry):
        zslice_v[pl.ds(i * L, L)] = jnp.zeros((L,), _f32)
        return carry

    lax.fori_loop(0, nslice // L, zbody, 0)
    pltpu.sync_copy(zslice_v, deg_sh.at[pl.ds(s * nslice, nslice)])
    plsc.subcore_barrier()

    nch = E // CH
    nt = NC * NS
    tid = c * NS + s

    def stage(ci, buf, sem):
        pltpu.async_copy(ei_hbm.at[1, pl.ds(ci * CH, CH)], buf, sem)

    def dr(buf, sem):
        pltpu.make_async_copy(ei_hbm.at[1, pl.ds(0, CH)], buf, sem).wait()

    def scat(buf):
        pltpu.sync_copy(ones_v, deg_sh.at[buf], add=True)

    @pl.when(tid < nch)
    def _():
        stage(tid, idxA, semA)

    def body(p, carry):
        i0 = tid + (2 * p) * nt
        i1 = i0 + nt
        i2 = i1 + nt

        @pl.when(i0 < nch)
        def _():
            @pl.when(i1 < nch)
            def _():
                stage(i1, idxB, semB)

            dr(idxA, semA)
            scat(idxA)

            @pl.when(i2 < nch)
            def _():
                stage(i2, idxA, semA)

            @pl.when(i1 < nch)
            def _():
                dr(idxB, semB)
                scat(idxB)

        return carry

    lax.fori_loop(0, (nch + 2 * nt - 1) // (2 * nt), body, 0)
    plsc.subcore_barrier()

    @pl.when(c == 0)
    def _():
        pltpu.sync_copy(
            deg_sh.at[pl.ds(s * nslice, nslice)],
            dega_hbm.at[pl.ds(s * nslice, nslice)],
        )

    @pl.when(c == 1)
    def _():
        pltpu.sync_copy(
            deg_sh.at[pl.ds(s * nslice, nslice)],
            degb_hbm.at[pl.ds(s * nslice, nslice)],
        )


def _agg_body(xws_hbm, ei_hbm, out_hbm,
              sstage, dstage, dst_v, rows0, rows1, z16_v, semA, semB, acc_sh):
    c = lax.axis_index("c")
    s = lax.axis_index("s")

    for i in range(16):
        for h in range(2):
            for j in range(128 // L):
                z16_v[i, h, pl.ds(j * L, L)] = jnp.zeros((L,), _f32)

    rpt = ACC_ROWS // NS

    def zb(i, carry):
        pltpu.sync_copy(z16_v, acc_sh.at[pl.ds(s * rpt + i * 16, 16)])
        return carry

    lax.fori_loop(0, rpt // 16, zb, 0)
    plsc.subcore_barrier()

    lo = c * HALF
    iota16 = lax.iota(jnp.int32, 16)

    def prep_segment(base, seglen):
        pltpu.sync_copy(ei_hbm.at[0, pl.ds(base, seglen)], sstage.at[pl.ds(0, seglen)])
        pltpu.sync_copy(ei_hbm.at[1, pl.ds(base, seglen)], dstage.at[pl.ds(0, seglen)])

        def tb(t, cnt):
            sl = pl.ds(t * L, L)
            sv = sstage[sl]
            dv = dstage[sl]
            m = (dv >= lo) & (dv < lo + HALF)
            plsc.store_compressed(sstage.at[pl.ds(cnt, L)], sv, mask=m)
            plsc.store_compressed(dstage.at[pl.ds(cnt, L)], dv - lo, mask=m)
            return cnt + plsc.all_reduce_population_count(m)[0]

        cnt = lax.fori_loop(0, seglen // L, tb, jnp.int32(0))

        def fb(k, carry):
            off = cnt + k * L
            sent = (k % 8) * L + iota16
            sstage[pl.ds(off, L)] = N + sent
            dstage[pl.ds(off, L)] = HALF + sent
            return carry

        lax.fori_loop(0, 2 * (ACH // L), fb, 0)
        return cnt

    def gather(k, rows, sem):
        pltpu.async_copy(xws_hbm.at[sstage.at[pl.ds(k * ACH, ACH)]], rows, sem)

    def drain(rows, sem):
        pltpu.make_async_copy(xws_hbm.at[sstage.at[pl.ds(0, ACH)]], rows, sem).wait()

    def scatter(k, rows):
        for j in range(ACH // L):
            dst_v[pl.ds(j * L, L)] = dstage[pl.ds(k * ACH + j * L, L)]
        pltpu.sync_copy(rows, acc_sh.at[dst_v], add=True)

    def seg_run(base, seglen):
        cnt = prep_segment(base, seglen)
        nch = (cnt + ACH - 1) // ACH
        npairs = (nch + 1) // 2

        @pl.when(npairs > 0)
        def _():
            gather(0, rows0, semA)

        def outer(g2, carry2):
            k0 = 2 * g2
            gather(k0 + 1, rows1, semB)
            drain(rows0, semA)
            scatter(k0, rows0)

            @pl.when(g2 + 1 < npairs)
            def _():
                gather(k0 + 2, rows0, semA)

            drain(rows1, semB)
            scatter(k0 + 1, rows1)
            return carry2

        lax.fori_loop(0, npairs, outer, 0)

    nseg = E // SEG

    def seg_body(i, carry):
        si = s + i * NS

        @pl.when(si < nseg)
        def _():
            seg_run(si * SEG, SEG)

        return carry

    lax.fori_loop(0, (nseg + NS - 1) // NS, seg_body, 0)

    @pl.when(s == nseg % NS)
    def _():
        seg_run(nseg * SEG, E - nseg * SEG)

    plsc.subcore_barrier()

    wpt = HALF // NS
    pltpu.sync_copy(
        acc_sh.at[pl.ds(s * wpt, wpt)],
        out_hbm.at[pl.ds(c * HALF + s * wpt, wpt)],
    )


@functools.cache
def _sc_kernels():
    mesh = plsc.VectorSubcoreMesh(
        core_axis_name="c", subcore_axis_name="s", num_cores=NC, num_subcores=NS
    )
    deg_kernel = pl.kernel(
        _deg_body,
        out_type=(
            jax.ShapeDtypeStruct((NP,), _f32),
            jax.ShapeDtypeStruct((NP,), _f32),
        ),
        mesh=mesh,
        scratch_types=[
            pltpu.VMEM((CH,), jnp.int32),
            pltpu.VMEM((CH,), jnp.int32),
            pltpu.VMEM((CH,), _f32),
            pltpu.VMEM((NP // NS,), _f32),
            pltpu.SemaphoreType.DMA,
            pltpu.SemaphoreType.DMA,
            pltpu.VMEM_SHARED((NP,), _f32),
        ],
    )
    agg_kernel = pl.kernel(
        _agg_body,
        out_type=jax.ShapeDtypeStruct((NP, 2, 128), _f32),
        mesh=mesh,
        compiler_params=pltpu.CompilerParams(needs_layout_passes=False),
        scratch_types=[
            pltpu.VMEM((SEG + 2 * ACH,), jnp.int32),
            pltpu.VMEM((SEG + 2 * ACH,), jnp.int32),
            pltpu.VMEM((ACH,), jnp.int32),
            pltpu.VMEM((ACH, 2, 128), _f32),
            pltpu.VMEM((ACH, 2, 128), _f32),
            pltpu.VMEM((16, 2, 128), _f32),
            pltpu.SemaphoreType.DMA,
            pltpu.SemaphoreType.DMA,
            pltpu.VMEM_SHARED((ACC_ROWS, 2, 128), _f32),
        ],
    )
    return deg_kernel, agg_kernel


def _gcn_mm_body(x_ref, w_ref, dega_ref, degb_ref, xw_ref, xws_ref):
    xw = jnp.dot(x_ref[...], w_ref[...], preferred_element_type=_f32)
    deg = dega_ref[...] + degb_ref[...] + 1.0
    dinv = lax.rsqrt(deg)
    zpad = jnp.zeros((NP - N, T), _f32)
    xw_ref[...] = jnp.concatenate([xw, zpad], axis=0)
    xws_ref[...] = jnp.concatenate([xw * dinv[0:N], zpad], axis=0)


def _mlp1_body(agg_ref, xw_ref, dega_ref, degb_ref, bg_ref, w1_ref, b1_ref, out_ref):
    deg = dega_ref[...] + degb_ref[...] + 1.0
    dinv = lax.rsqrt(deg)
    h = dinv * agg_ref[...] + (dinv * dinv) * xw_ref[...] + bg_ref[...]
    m1 = jnp.dot(h, w1_ref[...], preferred_element_type=_f32) + b1_ref[...]
    out_ref[...] = jnp.maximum(m1, 0.0)


def _mlp2_body(z_ref, w2_ref, b2_ref, out_ref):
    out_ref[...] = (
        jnp.dot(z_ref[...], w2_ref[...], preferred_element_type=_f32) + b2_ref[...]
    )


def kernel(x, edge_index, W_gcn, b_gcn, W1, b1, W2, b2):
    ei = edge_index.astype(jnp.int32)

    deg_kernel, agg_kernel = _sc_kernels()
    dega, degb = deg_kernel(ei)
    dega2 = dega.reshape(NP, 1)
    degb2 = degb.reshape(NP, 1)

    xw, xws = pl.pallas_call(
        _gcn_mm_body,
        out_shape=(
            jax.ShapeDtypeStruct((NP, T), _f32),
            jax.ShapeDtypeStruct((NP, T), _f32),
        ),
    )(x, W_gcn, dega2, degb2)

    agg = agg_kernel(xws.reshape(NP, 2, 128), ei).reshape(NP, T)

    m1 = pl.pallas_call(
        _mlp1_body,
        out_shape=jax.ShapeDtypeStruct((NP, H), _f32),
    )(agg, xw, dega2, degb2, b_gcn.reshape(1, T), W1, b1.reshape(1, H))

    z = m1[:N].reshape(N // H, H * H)

    y = pl.pallas_call(
        _mlp2_body,
        out_shape=jax.ShapeDtypeStruct((N // H, H * H), _f32),
    )(z, W2, b2.reshape(1, H * H))

    return y.reshape(-1)
